# baseline (speedup 1.0000x reference)
import functools

import jax
import jax.numpy as jnp
from jax import lax
from jax.experimental import pallas as pl
from jax.experimental.pallas import tpu as pltpu
from jax.experimental.pallas import tpu_sc as plsc

N_USERS = 25000
N_ITEMS = 25000
N_NODES = N_USERS + N_ITEMS
N_PAD = 50048
HALF = 32
N_EDGES = 800000
CHUNK = 512
EDGE_ROWS = 1568
E_PAD = EDGE_ROWS * CHUNK
REC = 3 * CHUNK
BATCH = 16384

NSUB = 16
ROWS_PER_TILE = EDGE_ROWS // NSUB
SLAB = N_PAD // NSUB
BIDX_ROWS = BATCH // CHUNK
BROWS_PER_TILE = BIDX_ROWS // NSUB
NGRP = CHUNK // 16


def _scale_rows(msg_ref, w_ref):
    def grp(g):
        w16 = w_ref[pl.ds(g * 16, 16)]
        for lane in range(16):
            e = g * 16 + lane
            wv = jnp.zeros((16,), jnp.float32) + w16[lane]
            wb32 = plsc.pack(wv, wv, format=plsc.PackFormat.INTERLEAVED)
            msg_ref[e, :] = msg_ref[e, :] * wb32
    pl.loop(0, NGRP, unroll=2)(grp)


def _unpack_add(dst_ref, src_ref):
    def it(r):
        a, b = plsc.unpack(src_ref[r, :], format=plsc.PackFormat.INTERLEAVED)
        dst_ref[r, pl.ds(0, 16)] = dst_ref[r, pl.ds(0, 16)] + a
        dst_ref[r, pl.ds(16, 16)] = dst_ref[r, pl.ds(16, 16)] + b
    pl.loop(0, CHUNK, unroll=4)(it)


def _unpack_set(dst_ref, src_ref):
    def it(r):
        a, b = plsc.unpack(src_ref[r, :], format=plsc.PackFormat.INTERLEAVED)
        dst_ref[r, pl.ds(0, 16)] = a
        dst_ref[r, pl.ds(16, 16)] = b
    pl.loop(0, CHUNK, unroll=4)(it)


def _gcn_body(e0, edata, uidx, iidx, zeros,
              part, e1, e2, e3,
              acc, ebuf0, ebuf1, ebuf2, ebuf3, srcb0, srcb1, srcb2, srcb3,
              dstb0, dstb1, dstb2, dstb3, wb0, wb1, wb2, wb3,
              msg0, msg1, msg2, msg3, ubff, ibff, resb,
              ge0, ge1, ge2, ge3, gg0, gg1, gg2, gg3, gs0, gs1, gs2, gs3):
    c = lax.axis_index("c")
    s = lax.axis_index("s")
    cbase = c * N_PAD
    base = s * ROWS_PER_TILE
    T = ROWS_PER_TILE

    ebuf = (ebuf0, ebuf1, ebuf2, ebuf3)
    srcb = (srcb0, srcb1, srcb2, srcb3)
    dstb = (dstb0, dstb1, dstb2, dstb3)
    wb = (wb0, wb1, wb2, wb3)
    msg = (msg0, msg1, msg2, msg3)
    esem = (ge0, ge1, ge2, ge3)
    gsem = (gg0, gg1, gg2, gg3)
    ssem = (gs0, gs1, gs2, gs3)

    def issue_eload(j, q):
        pltpu.async_copy(edata.at[pl.ds((base + j) * REC, REC)], ebuf[q], esem[q])

    def wait_eload(q):
        pltpu.make_async_copy(edata.at[pl.ds(0, REC)], ebuf[q], esem[q]).wait()

    def prep(q):
        def ext(g):
            sl = pl.ds(g * 16, 16)
            srcb[q][sl] = ebuf[q][pl.ds(g * 16, 16)].astype(jnp.int32) + cbase
            dstb[q][sl] = ebuf[q][pl.ds(CHUNK + g * 16, 16)].astype(jnp.int32)
            wb[q][sl] = ebuf[q][pl.ds(2 * CHUNK + g * 16, 16)]
        pl.loop(0, NGRP)(ext)

    for (cur, nxt) in ((e0, e1), (e1, e2), (e2, e3)):
        zdesc = pltpu.async_copy(zeros, acc.at[pl.ds(s * SLAB, SLAB)], gs1)

        def wait_gather(q, cur=cur):
            pltpu.make_async_copy(cur.at[pl.ds(0, CHUNK)], msg[q], gsem[q]).wait()

        def issue_gather(q, cur=cur):
            pltpu.async_copy(cur.at[srcb[q]], msg[q], gsem[q])

        def wait_scatter(q):
            pltpu.make_async_copy(msg[q], acc.at[pl.ds(0, CHUNK)], ssem[q]).wait()

        def issue_scatter(q):
            pltpu.async_copy(msg[q], acc.at[dstb[q]], ssem[q], add=True)

        def edge_iter(j, q, first):
            q3 = (q + 3) % 4
            has_j3 = not isinstance(j, int) or j + 3 <= T - 1
            has_j4 = not isinstance(j, int) or j + 4 <= T - 1
            if has_j3:
                wait_eload(q3)
            if not first:
                wait_scatter(q3)
            if has_j3:
                prep(q3)
            if has_j4:
                issue_eload(j + 4, q)
            if has_j3:
                issue_gather(q3)
            wait_gather(q)
            _scale_rows(msg[q], wb[q])
            issue_scatter(q)

        for p in range(3):
            pltpu.sync_copy(edata.at[pl.ds((base + p) * REC, REC)], ebuf[p])
            prep(p)
        issue_gather(0)
        issue_gather(1)
        issue_gather(2)
        issue_eload(3, 3)
        zdesc.wait()
        plsc.subcore_barrier()

        edge_iter(0, 0, True)
        def steady(jd):
            edge_iter(jd, 1, False)
            edge_iter(jd + 1, 2, False)
            edge_iter(jd + 2, 3, False)
            edge_iter(jd + 3, 0, False)
        S = 1 + 4 * ((T - 5) // 4)
        pl.loop(1, S, step=4)(steady)
        for j in range(S, T):
            edge_iter(j, j % 4, False)
        wait_scatter((T - 1) % 4)

        plsc.subcore_barrier()
        pltpu.sync_copy(acc.at[pl.ds(s * SLAB, SLAB)],
                        nxt.at[pl.ds(cbase + s * SLAB, SLAB)])
        plsc.subcore_barrier()

    def bchunk(k):
        row = s * BROWS_PER_TILE + k
        for idx_hbm, fbuf, q in ((uidx, ubff, 0), (iidx, ibff, 1)):
            pltpu.sync_copy(idx_hbm.at[pl.ds(row * CHUNK, CHUNK)], dstb[q])
            def adj(g):
                sl = pl.ds(g * 16, 16)
                srcb[q][sl] = dstb[q][sl] + cbase
            pl.loop(0, NGRP)(adj)
            descs = [pltpu.async_copy(lay.at[srcb[q]], msg[l], gsem[l])
                     for l, lay in enumerate((e0, e1, e2, e3))]
            descs[0].wait()
            _unpack_set(fbuf, msg[0])
            for l in (1, 2, 3):
                descs[l].wait()
                _unpack_add(fbuf, msg[l])
        def dot(g):
            erow = lax.iota(jnp.int32, 16) + g * 16
            def dd(d, acc16):
                dsp = jnp.zeros((16,), jnp.int32) + d
                uv = plsc.load_gather(ubff, [erow, dsp])
                iv = plsc.load_gather(ibff, [erow, dsp])
                return acc16 + uv * iv
            acc16 = pl.loop(0, HALF, init_carry=jnp.zeros((16,), jnp.float32),
                            unroll=8)(dd)
            resb[pl.ds(g * 16, 16)] = acc16 * jnp.float32(1.0 / 16.0)
        pl.loop(0, NGRP)(dot)
        pltpu.sync_copy(resb, part.at[pl.ds(c * BATCH + row * CHUNK, CHUNK)])
    pl.loop(0, BROWS_PER_TILE)(bchunk)


@functools.partial(jax.jit, static_argnums=())
def _run(e0, edata, uidx, iidx, zeros):
    mesh = plsc.VectorSubcoreMesh(core_axis_name="c", subcore_axis_name="s")
    f = pl.kernel(
        _gcn_body,
        out_type=(
            jax.ShapeDtypeStruct((2 * BATCH,), jnp.float32),
            jax.ShapeDtypeStruct((2 * N_PAD, HALF), jnp.bfloat16),
            jax.ShapeDtypeStruct((2 * N_PAD, HALF), jnp.bfloat16),
            jax.ShapeDtypeStruct((2 * N_PAD, HALF), jnp.bfloat16),
        ),
        mesh=mesh,
        scratch_types=(
            pltpu.VMEM_SHARED((N_PAD, HALF), jnp.bfloat16),
            pltpu.VMEM((REC,), jnp.float32),
            pltpu.VMEM((REC,), jnp.float32),
            pltpu.VMEM((REC,), jnp.float32),
            pltpu.VMEM((REC,), jnp.float32),
            pltpu.VMEM((CHUNK,), jnp.int32),
            pltpu.VMEM((CHUNK,), jnp.int32),
            pltpu.VMEM((CHUNK,), jnp.int32),
            pltpu.VMEM((CHUNK,), jnp.int32),
            pltpu.VMEM((CHUNK,), jnp.int32),
            pltpu.VMEM((CHUNK,), jnp.int32),
            pltpu.VMEM((CHUNK,), jnp.int32),
            pltpu.VMEM((CHUNK,), jnp.int32),
            pltpu.VMEM((CHUNK,), jnp.float32),
            pltpu.VMEM((CHUNK,), jnp.float32),
            pltpu.VMEM((CHUNK,), jnp.float32),
            pltpu.VMEM((CHUNK,), jnp.float32),
            pltpu.VMEM((CHUNK, HALF), jnp.bfloat16),
            pltpu.VMEM((CHUNK, HALF), jnp.bfloat16),
            pltpu.VMEM((CHUNK, HALF), jnp.bfloat16),
            pltpu.VMEM((CHUNK, HALF), jnp.bfloat16),
            pltpu.VMEM((CHUNK, HALF), jnp.float32),
            pltpu.VMEM((CHUNK, HALF), jnp.float32),
            pltpu.VMEM((CHUNK,), jnp.float32),
            pltpu.SemaphoreType.DMA,
            pltpu.SemaphoreType.DMA,
            pltpu.SemaphoreType.DMA,
            pltpu.SemaphoreType.DMA,
            pltpu.SemaphoreType.DMA,
            pltpu.SemaphoreType.DMA,
            pltpu.SemaphoreType.DMA,
            pltpu.SemaphoreType.DMA,
            pltpu.SemaphoreType.DMA,
            pltpu.SemaphoreType.DMA,
            pltpu.SemaphoreType.DMA,
            pltpu.SemaphoreType.DMA,
        ),
        compiler_params=pltpu.CompilerParams(
            use_tc_tiling_on_sc=False, needs_layout_passes=False),
    )
    return f(e0, edata, uidx, iidx, zeros)


def kernel(user_table, item_table, edge_weight, edge_src, edge_dst, user_idx, item_idx):
    all_emb = jnp.concatenate([user_table, item_table], axis=0)
    all_emb = jnp.pad(all_emb, ((0, N_PAD - N_NODES), (0, 0)))
    e0 = all_emb.reshape(N_PAD, 2, HALF).transpose(1, 0, 2).reshape(2 * N_PAD, HALF)
    e0 = e0.astype(jnp.bfloat16)
    pad = E_PAD - N_EDGES
    src_p = jnp.concatenate([edge_src, jnp.zeros((pad,), jnp.int32)])
    dst_p = jnp.concatenate([edge_dst, jnp.zeros((pad,), jnp.int32)])
    w_p = jnp.concatenate([edge_weight, jnp.zeros((pad,), jnp.float32)])
    edata = jnp.stack(
        [src_p.reshape(EDGE_ROWS, CHUNK).astype(jnp.float32),
         dst_p.reshape(EDGE_ROWS, CHUNK).astype(jnp.float32),
         w_p.reshape(EDGE_ROWS, CHUNK)],
        axis=1).reshape(-1)
    uidx = user_idx
    iidx = item_idx + N_USERS
    zeros = jnp.zeros((SLAB, HALF), jnp.bfloat16)
    part, _, _, _ = _run(e0, edata, uidx, iidx, zeros)
    return part[:BATCH] + part[BATCH:]


# speedup vs baseline: 1.0058x; 1.0058x over previous
import functools

import jax
import jax.numpy as jnp
from jax import lax
from jax.experimental import pallas as pl
from jax.experimental.pallas import tpu as pltpu
from jax.experimental.pallas import tpu_sc as plsc

N_USERS = 25000
N_ITEMS = 25000
N_NODES = N_USERS + N_ITEMS
N_PAD = 50048
HALF = 32
N_EDGES = 800000
CHUNK = 512
EDGE_ROWS = 1568
E_PAD = EDGE_ROWS * CHUNK
REC = 3 * CHUNK
BATCH = 16384

NSUB = 16
ROWS_PER_TILE = EDGE_ROWS // NSUB
SLAB = N_PAD // NSUB
BIDX_ROWS = BATCH // CHUNK
BROWS_PER_TILE = BIDX_ROWS // NSUB
NGRP = CHUNK // 16


def _scale_rows(msg_ref, w_ref):
    def grp(g):
        w16 = w_ref[pl.ds(g * 16, 16)]
        for lane in range(16):
            e = g * 16 + lane
            wv = jnp.zeros((16,), jnp.float32) + w16[lane]
            wb32 = plsc.pack(wv, wv, format=plsc.PackFormat.INTERLEAVED)
            msg_ref[e, :] = msg_ref[e, :] * wb32
    pl.loop(0, NGRP)(grp)


def _unpack_add(dst_ref, src_ref):
    def it(r):
        a, b = plsc.unpack(src_ref[r, :], format=plsc.PackFormat.INTERLEAVED)
        dst_ref[r, pl.ds(0, 16)] = dst_ref[r, pl.ds(0, 16)] + a
        dst_ref[r, pl.ds(16, 16)] = dst_ref[r, pl.ds(16, 16)] + b
    pl.loop(0, CHUNK, unroll=4)(it)


def _unpack_set(dst_ref, src_ref):
    def it(r):
        a, b = plsc.unpack(src_ref[r, :], format=plsc.PackFormat.INTERLEAVED)
        dst_ref[r, pl.ds(0, 16)] = a
        dst_ref[r, pl.ds(16, 16)] = b
    pl.loop(0, CHUNK, unroll=4)(it)


def _gcn_body(e0, edata, uidx, iidx, zeros,
              part, e1, e2, e3,
              acc, ebuf0, ebuf1, ebuf2, ebuf3, srcb0, srcb1, srcb2, srcb3,
              dstb0, dstb1, dstb2, dstb3, wb0, wb1, wb2, wb3,
              msg0, msg1, msg2, msg3, ubff, ibff, resb,
              ge0, ge1, ge2, ge3, gg0, gg1, gg2, gg3, gs0, gs1, gs2, gs3):
    c = lax.axis_index("c")
    s = lax.axis_index("s")
    cbase = c * N_PAD
    base = s * ROWS_PER_TILE
    T = ROWS_PER_TILE

    ebuf = (ebuf0, ebuf1, ebuf2, ebuf3)
    srcb = (srcb0, srcb1, srcb2, srcb3)
    dstb = (dstb0, dstb1, dstb2, dstb3)
    wb = (wb0, wb1, wb2, wb3)
    msg = (msg0, msg1, msg2, msg3)
    esem = (ge0, ge1, ge2, ge3)
    gsem = (gg0, gg1, gg2, gg3)
    ssem = (gs0, gs1, gs2, gs3)

    def issue_eload(j, q):
        pltpu.async_copy(edata.at[pl.ds((base + j) * REC, REC)], ebuf[q], esem[q])

    def wait_eload(q):
        pltpu.make_async_copy(edata.at[pl.ds(0, REC)], ebuf[q], esem[q]).wait()

    def prep(q):
        def ext(g):
            sl = pl.ds(g * 16, 16)
            srcb[q][sl] = ebuf[q][pl.ds(g * 16, 16)].astype(jnp.int32) + cbase
            dstb[q][sl] = ebuf[q][pl.ds(CHUNK + g * 16, 16)].astype(jnp.int32)
            wb[q][sl] = ebuf[q][pl.ds(2 * CHUNK + g * 16, 16)]
        pl.loop(0, NGRP)(ext)

    for (cur, nxt) in ((e0, e1), (e1, e2), (e2, e3)):
        zdesc = pltpu.async_copy(zeros, acc.at[pl.ds(s * SLAB, SLAB)], gs1)

        def wait_gather(q, cur=cur):
            pltpu.make_async_copy(cur.at[pl.ds(0, CHUNK)], msg[q], gsem[q]).wait()

        def issue_gather(q, cur=cur):
            pltpu.async_copy(cur.at[srcb[q]], msg[q], gsem[q])

        def wait_scatter(q):
            pltpu.make_async_copy(msg[q], acc.at[pl.ds(0, CHUNK)], ssem[q]).wait()

        def issue_scatter(q):
            pltpu.async_copy(msg[q], acc.at[dstb[q]], ssem[q], add=True)

        def edge_iter(j, q, first):
            q3 = (q + 3) % 4
            has_j3 = not isinstance(j, int) or j + 3 <= T - 1
            has_j4 = not isinstance(j, int) or j + 4 <= T - 1
            if has_j3:
                wait_eload(q3)
            if not first:
                wait_scatter(q3)
            if has_j3:
                prep(q3)
            if has_j4:
                issue_eload(j + 4, q)
            if has_j3:
                issue_gather(q3)
            wait_gather(q)
            _scale_rows(msg[q], wb[q])
            issue_scatter(q)

        for p in range(3):
            pltpu.sync_copy(edata.at[pl.ds((base + p) * REC, REC)], ebuf[p])
            prep(p)
        issue_gather(0)
        issue_gather(1)
        issue_gather(2)
        issue_eload(3, 3)
        zdesc.wait()
        plsc.subcore_barrier()

        edge_iter(0, 0, True)
        def steady(jd):
            edge_iter(jd, 1, False)
            edge_iter(jd + 1, 2, False)
            edge_iter(jd + 2, 3, False)
            edge_iter(jd + 3, 0, False)
        S = 1 + 4 * ((T - 5) // 4)
        pl.loop(1, S, step=4)(steady)
        for j in range(S, T):
            edge_iter(j, j % 4, False)
        wait_scatter((T - 1) % 4)

        plsc.subcore_barrier()
        pltpu.sync_copy(acc.at[pl.ds(s * SLAB, SLAB)],
                        nxt.at[pl.ds(cbase + s * SLAB, SLAB)])
        plsc.subcore_barrier()

    def bchunk(k):
        row = s * BROWS_PER_TILE + k
        for idx_hbm, fbuf, q in ((uidx, ubff, 0), (iidx, ibff, 1)):
            pltpu.sync_copy(idx_hbm.at[pl.ds(row * CHUNK, CHUNK)], dstb[q])
            def adj(g):
                sl = pl.ds(g * 16, 16)
                srcb[q][sl] = dstb[q][sl] + cbase
            pl.loop(0, NGRP)(adj)
            descs = [pltpu.async_copy(lay.at[srcb[q]], msg[l], gsem[l])
                     for l, lay in enumerate((e0, e1, e2, e3))]
            descs[0].wait()
            _unpack_set(fbuf, msg[0])
            for l in (1, 2, 3):
                descs[l].wait()
                _unpack_add(fbuf, msg[l])
        def dot(g):
            erow = lax.iota(jnp.int32, 16) + g * 16
            def dd(d, acc16):
                dsp = jnp.zeros((16,), jnp.int32) + d
                uv = plsc.load_gather(ubff, [erow, dsp])
                iv = plsc.load_gather(ibff, [erow, dsp])
                return acc16 + uv * iv
            acc16 = pl.loop(0, HALF, init_carry=jnp.zeros((16,), jnp.float32),
                            unroll=8)(dd)
            resb[pl.ds(g * 16, 16)] = acc16 * jnp.float32(1.0 / 16.0)
        pl.loop(0, NGRP)(dot)
        pltpu.sync_copy(resb, part.at[pl.ds(c * BATCH + row * CHUNK, CHUNK)])
    pl.loop(0, BROWS_PER_TILE)(bchunk)


@functools.partial(jax.jit, static_argnums=())
def _run(e0, edata, uidx, iidx, zeros):
    mesh = plsc.VectorSubcoreMesh(core_axis_name="c", subcore_axis_name="s")
    f = pl.kernel(
        _gcn_body,
        out_type=(
            jax.ShapeDtypeStruct((2 * BATCH,), jnp.float32),
            jax.ShapeDtypeStruct((2 * N_PAD, HALF), jnp.bfloat16),
            jax.ShapeDtypeStruct((2 * N_PAD, HALF), jnp.bfloat16),
            jax.ShapeDtypeStruct((2 * N_PAD, HALF), jnp.bfloat16),
        ),
        mesh=mesh,
        scratch_types=(
            pltpu.VMEM_SHARED((N_PAD, HALF), jnp.bfloat16),
            pltpu.VMEM((REC,), jnp.float32),
            pltpu.VMEM((REC,), jnp.float32),
            pltpu.VMEM((REC,), jnp.float32),
            pltpu.VMEM((REC,), jnp.float32),
            pltpu.VMEM((CHUNK,), jnp.int32),
            pltpu.VMEM((CHUNK,), jnp.int32),
            pltpu.VMEM((CHUNK,), jnp.int32),
            pltpu.VMEM((CHUNK,), jnp.int32),
            pltpu.VMEM((CHUNK,), jnp.int32),
            pltpu.VMEM((CHUNK,), jnp.int32),
            pltpu.VMEM((CHUNK,), jnp.int32),
            pltpu.VMEM((CHUNK,), jnp.int32),
            pltpu.VMEM((CHUNK,), jnp.float32),
            pltpu.VMEM((CHUNK,), jnp.float32),
            pltpu.VMEM((CHUNK,), jnp.float32),
            pltpu.VMEM((CHUNK,), jnp.float32),
            pltpu.VMEM((CHUNK, HALF), jnp.bfloat16),
            pltpu.VMEM((CHUNK, HALF), jnp.bfloat16),
            pltpu.VMEM((CHUNK, HALF), jnp.bfloat16),
            pltpu.VMEM((CHUNK, HALF), jnp.bfloat16),
            pltpu.VMEM((CHUNK, HALF), jnp.float32),
            pltpu.VMEM((CHUNK, HALF), jnp.float32),
            pltpu.VMEM((CHUNK,), jnp.float32),
            pltpu.SemaphoreType.DMA,
            pltpu.SemaphoreType.DMA,
            pltpu.SemaphoreType.DMA,
            pltpu.SemaphoreType.DMA,
            pltpu.SemaphoreType.DMA,
            pltpu.SemaphoreType.DMA,
            pltpu.SemaphoreType.DMA,
            pltpu.SemaphoreType.DMA,
            pltpu.SemaphoreType.DMA,
            pltpu.SemaphoreType.DMA,
            pltpu.SemaphoreType.DMA,
            pltpu.SemaphoreType.DMA,
        ),
        compiler_params=pltpu.CompilerParams(
            use_tc_tiling_on_sc=False, needs_layout_passes=False),
    )
    return f(e0, edata, uidx, iidx, zeros)


def kernel(user_table, item_table, edge_weight, edge_src, edge_dst, user_idx, item_idx):
    all_emb = jnp.concatenate([user_table, item_table], axis=0)
    all_emb = jnp.pad(all_emb, ((0, N_PAD - N_NODES), (0, 0)))
    e0 = all_emb.reshape(N_PAD, 2, HALF).transpose(1, 0, 2).reshape(2 * N_PAD, HALF)
    e0 = e0.astype(jnp.bfloat16)
    pad = E_PAD - N_EDGES
    src_p = jnp.concatenate([edge_src, jnp.zeros((pad,), jnp.int32)])
    dst_p = jnp.concatenate([edge_dst, jnp.zeros((pad,), jnp.int32)])
    w_p = jnp.concatenate([edge_weight, jnp.zeros((pad,), jnp.float32)])
    edata = jnp.stack(
        [src_p.reshape(EDGE_ROWS, CHUNK).astype(jnp.float32),
         dst_p.reshape(EDGE_ROWS, CHUNK).astype(jnp.float32),
         w_p.reshape(EDGE_ROWS, CHUNK)],
        axis=1).reshape(-1)
    uidx = user_idx
    iidx = item_idx + N_USERS
    zeros = jnp.zeros((SLAB, HALF), jnp.bfloat16)
    part, _, _, _ = _run(e0, edata, uidx, iidx, zeros)
    return part[:BATCH] + part[BATCH:]
